# baseline (speedup 1.0000x reference)
import functools

import jax
import jax.numpy as jnp
from jax import lax
from jax.experimental import pallas as pl
from jax.experimental.pallas import tpu as pltpu
from jax.experimental.pallas import tpu_sc as plsc

VOCAB = 100000
DIM = 128
SEQ = 200
BATCH = 1024

NC = 2
NS = 16
NW = NC * NS
L = 16
NJ = DIM // L

TOTAL = BATCH * SEQ
CHUNK = 128
NCHUNKS = TOTAL // CHUNK
CPW = NCHUNKS // NW
EPS = 1e-12


def _rsqrt(x):
    i = plsc.bitcast(x, jnp.int32)
    i = jnp.int32(0x5F3759DF) - (i >> 1)
    y = plsc.bitcast(i, jnp.float32)
    half = x * jnp.float32(0.5)
    for _ in range(2):
        y = y * (jnp.float32(1.5) - half * y * y)
    return y


def _emb_body(comb_hbm, tok_hbm, pos_hbm, segtab_hbm, gam_hbm, bet_hbm,
              out_hbm, comb_v0, comb_v1, rows_v0, rows_v1,
              out_v, pos_v, segtab_v, gam_v, bet_v, gsem0, gsem1, ssem):
    wid = lax.axis_index("s") * NC + lax.axis_index("c")
    comb_v = [comb_v0, comb_v1]
    rows_v = [rows_v0, rows_v1]
    gsem = [gsem0, gsem1]

    pltpu.sync_copy(pos_hbm, pos_v.at[pl.ds(0, SEQ)])
    pltpu.sync_copy(pos_hbm, pos_v.at[pl.ds(SEQ, SEQ)])
    pltpu.sync_copy(segtab_hbm, segtab_v)
    pltpu.sync_copy(gam_hbm, gam_v)
    pltpu.sync_copy(bet_hbm, bet_v)

    def fold(t, carry):
        for j in range(NJ):
            sl = pl.ds(j * L, L)
            pos_v[t, sl] = pos_v[t, sl] + segtab_v[0, sl]
            pos_v[SEQ + t, sl] = pos_v[SEQ + t, sl] + segtab_v[1, sl]
        return carry

    lax.fori_loop(0, SEQ, fold, 0)

    gam = [gam_v[pl.ds(j * L, L)] for j in range(NJ)]
    bet = [bet_v[pl.ds(j * L, L)] for j in range(NJ)]

    inv_dim = jnp.float32(1.0 / DIM)
    zeros16 = jnp.zeros((L,), jnp.int32)

    def start_chunk(k, b):
        c = wid * CPW + k
        pltpu.sync_copy(comb_hbm.at[c], comb_v[b])
        return pltpu.async_copy(tok_hbm.at[comb_v[b].at[0]], rows_v[b],
                                gsem[b])

    def compute_chunk(k, b):
        c = wid * CPW + k
        base = c * CHUNK

        NU = 8
        NQ = 4
        QTOK = CHUNK // NQ
        ones16 = zeros16 + 1
        stores = []
        for q in range(NQ):
            qbase = q * QTOK

            def tok_body(i, tc, qbase=qbase):
                ts = [qbase + NU * i + u for u in range(NU)]
                xs = [[] for _ in range(NU)]
                stats = []
                for u in range(NU):
                    t = ts[u]
                    s = plsc.load_gather(comb_v[b], [ones16, zeros16 + t])[0]
                    p = lax.rem(base + t, SEQ) + s * SEQ
                    acc = None
                    acc2 = None
                    for j in range(NJ):
                        sl = pl.ds(j * L, L)
                        x = rows_v[b][t, sl] + pos_v[p, sl]
                        xs[u].append(x)
                        acc = x if acc is None else acc + x
                        acc2 = x * x if acc2 is None else acc2 + x * x
                    stats.append((acc, acc2))
                rm = []
                for u in range(NU):
                    acc, acc2 = stats[u]
                    mean = jnp.sum(acc) * inv_dim
                    var = jnp.maximum(
                        jnp.sum(acc2) * inv_dim - mean * mean, 0.0)
                    rvec = _rsqrt(
                        jnp.full((L,), var, jnp.float32) + jnp.float32(EPS))
                    mvec = jnp.full((L,), mean, jnp.float32)
                    rm.append((rvec, mvec))
                for u in range(NU):
                    rvec, mvec = rm[u]
                    for j in range(NJ):
                        sl = pl.ds(j * L, L)
                        out_v[ts[u], sl] = (
                            (xs[u][j] - mvec) * rvec * gam[j] + bet[j])
                return tc

            lax.fori_loop(0, QTOK // NU, tok_body, 0)
            stores.append(pltpu.async_copy(
                out_v.at[pl.ds(qbase, QTOK)],
                out_hbm.at[c, pl.ds(qbase, QTOK)], ssem))
        for d in stores:
            d.wait()

    start_chunk(0, 0).wait()

    def pair_body(i, carry):
        d1 = start_chunk(2 * i + 1, 1)
        compute_chunk(2 * i, 0)
        d1.wait()
        d0 = start_chunk(2 * i + 2, 0)
        compute_chunk(2 * i + 1, 1)
        d0.wait()
        return carry

    lax.fori_loop(0, CPW // 2 - 1, pair_body, 0)
    d1 = start_chunk(CPW - 1, 1)
    compute_chunk(CPW - 2, 0)
    d1.wait()
    compute_chunk(CPW - 1, 1)


@jax.jit
def _emb_call(comb, token_table, pos_table, seg_table, gamma, beta):
    mesh = plsc.VectorSubcoreMesh(core_axis_name="c", subcore_axis_name="s",
                                  num_cores=NC, num_subcores=NS)
    fn = pl.kernel(
        _emb_body,
        out_type=jax.ShapeDtypeStruct((NCHUNKS, CHUNK, DIM), jnp.float32),
        mesh=mesh,
        compiler_params=pltpu.CompilerParams(needs_layout_passes=False),
        scratch_types=[
            pltpu.VMEM((2, CHUNK), jnp.int32),
            pltpu.VMEM((2, CHUNK), jnp.int32),
            pltpu.VMEM((CHUNK, DIM), jnp.float32),
            pltpu.VMEM((CHUNK, DIM), jnp.float32),
            pltpu.VMEM((CHUNK, DIM), jnp.float32),
            pltpu.VMEM((2 * SEQ, DIM), jnp.float32),
            pltpu.VMEM((2, DIM), jnp.float32),
            pltpu.VMEM((DIM,), jnp.float32),
            pltpu.VMEM((DIM,), jnp.float32),
            pltpu.SemaphoreType.DMA,
            pltpu.SemaphoreType.DMA,
            pltpu.SemaphoreType.DMA,
        ],
    )
    return fn(comb, token_table, pos_table, seg_table, gamma, beta)


def kernel(input_ids, segment_ids, token_table, pos_table, seg_table, gamma, beta):
    ids2 = input_ids.reshape(NCHUNKS, CHUNK).astype(jnp.int32)
    seg2 = segment_ids.reshape(NCHUNKS, CHUNK).astype(jnp.int32)
    comb = jnp.stack([ids2, seg2], axis=1)
    out = _emb_call(comb, token_table, pos_table, seg_table, gamma, beta)
    return out.reshape(BATCH, SEQ, DIM)


# speedup vs baseline: 1.0225x; 1.0225x over previous
import functools

import jax
import jax.numpy as jnp
from jax import lax
from jax.experimental import pallas as pl
from jax.experimental.pallas import tpu as pltpu
from jax.experimental.pallas import tpu_sc as plsc

VOCAB = 100000
DIM = 128
SEQ = 200
BATCH = 1024

NC = 2
NS = 16
NW = NC * NS
L = 16
NJ = DIM // L

TOTAL = BATCH * SEQ
CHUNK = 128
NCHUNKS = TOTAL // CHUNK
CPW = NCHUNKS // NW
EPS = 1e-12


def _rsqrt(x):
    i = plsc.bitcast(x, jnp.int32)
    i = jnp.int32(0x5F3759DF) - (i >> 1)
    y = plsc.bitcast(i, jnp.float32)
    half = x * jnp.float32(0.5)
    for _ in range(2):
        y = y * (jnp.float32(1.5) - half * y * y)
    return y


def _emb_body(comb_hbm, tok_hbm, pos_hbm, segtab_hbm, gam_hbm, bet_hbm,
              out_hbm, comb_v0, comb_v1, rows_v0, rows_v1,
              out_v, pos_v, segtab_v, gam_v, bet_v, gsem0, gsem1, ssem):
    wid = lax.axis_index("s") * NC + lax.axis_index("c")
    comb_v = [comb_v0, comb_v1]
    rows_v = [rows_v0, rows_v1]
    gsem = [gsem0, gsem1]

    pltpu.sync_copy(pos_hbm, pos_v.at[pl.ds(0, SEQ)])
    pltpu.sync_copy(pos_hbm, pos_v.at[pl.ds(SEQ, SEQ)])
    pltpu.sync_copy(segtab_hbm, segtab_v)
    pltpu.sync_copy(gam_hbm, gam_v)
    pltpu.sync_copy(bet_hbm, bet_v)

    def fold(t, carry):
        for j in range(NJ):
            sl = pl.ds(j * L, L)
            pos_v[t, sl] = pos_v[t, sl] + segtab_v[0, sl]
            pos_v[SEQ + t, sl] = pos_v[SEQ + t, sl] + segtab_v[1, sl]
        return carry

    lax.fori_loop(0, SEQ, fold, 0)

    gam = [gam_v[pl.ds(j * L, L)] for j in range(NJ)]
    bet = [bet_v[pl.ds(j * L, L)] for j in range(NJ)]

    inv_dim = jnp.float32(1.0 / DIM)
    zeros16 = jnp.zeros((L,), jnp.int32)

    NG = 2
    GR = CHUNK // NG

    def start_chunk(k, b):
        c = wid * CPW + k
        pltpu.sync_copy(comb_hbm.at[c], comb_v[b])
        return [
            pltpu.async_copy(
                tok_hbm.at[comb_v[b].at[0, pl.ds(h * GR, GR)]],
                rows_v[b].at[pl.ds(h * GR, GR)], gsem[b])
            for h in range(NG)
        ]

    def compute_chunk(k, b):
        c = wid * CPW + k
        base = c * CHUNK

        NU = 4
        NQ = 4
        QTOK = CHUNK // NQ
        ones16 = zeros16 + 1
        stores = []
        for q in range(NQ):
            qbase = q * QTOK

            def tok_body(i, tc, qbase=qbase):
                ts = [qbase + NU * i + u for u in range(NU)]
                xs = [[] for _ in range(NU)]
                stats = []
                for u in range(NU):
                    t = ts[u]
                    s = plsc.load_gather(comb_v[b], [ones16, zeros16 + t])[0]
                    p = lax.rem(base + t, SEQ) + s * SEQ
                    acc = None
                    acc2 = None
                    for j in range(NJ):
                        sl = pl.ds(j * L, L)
                        x = rows_v[b][t, sl] + pos_v[p, sl]
                        xs[u].append(x)
                        acc = x if acc is None else acc + x
                        acc2 = x * x if acc2 is None else acc2 + x * x
                    stats.append((acc, acc2))
                rm = []
                for u in range(NU):
                    acc, acc2 = stats[u]
                    mean = jnp.sum(acc) * inv_dim
                    var = jnp.maximum(
                        jnp.sum(acc2) * inv_dim - mean * mean, 0.0)
                    rvec = _rsqrt(
                        jnp.full((L,), var, jnp.float32) + jnp.float32(EPS))
                    mvec = jnp.full((L,), mean, jnp.float32)
                    rm.append((rvec, mvec))
                for u in range(NU):
                    rvec, mvec = rm[u]
                    for j in range(NJ):
                        sl = pl.ds(j * L, L)
                        out_v[ts[u], sl] = (
                            (xs[u][j] - mvec) * rvec * gam[j] + bet[j])
                return tc

            lax.fori_loop(0, QTOK // NU, tok_body, 0)
            stores.append(pltpu.async_copy(
                out_v.at[pl.ds(qbase, QTOK)],
                out_hbm.at[c, pl.ds(qbase, QTOK)], ssem))
        for d in stores:
            d.wait()

    for d in start_chunk(0, 0):
        d.wait()

    def pair_body(i, carry):
        d1 = start_chunk(2 * i + 1, 1)
        compute_chunk(2 * i, 0)
        for d in d1:
            d.wait()
        d0 = start_chunk(2 * i + 2, 0)
        compute_chunk(2 * i + 1, 1)
        for d in d0:
            d.wait()
        return carry

    lax.fori_loop(0, CPW // 2 - 1, pair_body, 0)
    d1 = start_chunk(CPW - 1, 1)
    compute_chunk(CPW - 2, 0)
    for d in d1:
        d.wait()
    compute_chunk(CPW - 1, 1)


@jax.jit
def _emb_call(comb, token_table, pos_table, seg_table, gamma, beta):
    mesh = plsc.VectorSubcoreMesh(core_axis_name="c", subcore_axis_name="s",
                                  num_cores=NC, num_subcores=NS)
    fn = pl.kernel(
        _emb_body,
        out_type=jax.ShapeDtypeStruct((NCHUNKS, CHUNK, DIM), jnp.float32),
        mesh=mesh,
        compiler_params=pltpu.CompilerParams(needs_layout_passes=False),
        scratch_types=[
            pltpu.VMEM((2, CHUNK), jnp.int32),
            pltpu.VMEM((2, CHUNK), jnp.int32),
            pltpu.VMEM((CHUNK, DIM), jnp.float32),
            pltpu.VMEM((CHUNK, DIM), jnp.float32),
            pltpu.VMEM((CHUNK, DIM), jnp.float32),
            pltpu.VMEM((2 * SEQ, DIM), jnp.float32),
            pltpu.VMEM((2, DIM), jnp.float32),
            pltpu.VMEM((DIM,), jnp.float32),
            pltpu.VMEM((DIM,), jnp.float32),
            pltpu.SemaphoreType.DMA,
            pltpu.SemaphoreType.DMA,
            pltpu.SemaphoreType.DMA,
        ],
    )
    return fn(comb, token_table, pos_table, seg_table, gamma, beta)


def kernel(input_ids, segment_ids, token_table, pos_table, seg_table, gamma, beta):
    ids2 = input_ids.reshape(NCHUNKS, CHUNK).astype(jnp.int32)
    seg2 = segment_ids.reshape(NCHUNKS, CHUNK).astype(jnp.int32)
    comb = jnp.stack([ids2, seg2], axis=1)
    out = _emb_call(comb, token_table, pos_table, seg_table, gamma, beta)
    return out.reshape(BATCH, SEQ, DIM)


# speedup vs baseline: 1.1615x; 1.1359x over previous
import functools

import jax
import jax.numpy as jnp
from jax import lax
from jax.experimental import pallas as pl
from jax.experimental.pallas import tpu as pltpu
from jax.experimental.pallas import tpu_sc as plsc

VOCAB = 100000
DIM = 128
SEQ = 200
BATCH = 1024

NC = 2
NS = 16
NW = NC * NS
L = 16
NJ = DIM // L

TOTAL = BATCH * SEQ
CHUNK = 128
NCHUNKS = TOTAL // CHUNK
CPW = NCHUNKS // NW
EPS = 1e-12


def _rsqrt(x):
    i = plsc.bitcast(x, jnp.int32)
    i = jnp.int32(0x5F3759DF) - (i >> 1)
    y = plsc.bitcast(i, jnp.float32)
    half = x * jnp.float32(0.5)
    for _ in range(2):
        y = y * (jnp.float32(1.5) - half * y * y)
    return y


def _emb_body(comb_hbm, tok_hbm, pos_hbm, segtab_hbm, gam_hbm, bet_hbm,
              out_hbm, comb_v0, comb_v1, pidx_v0, pidx_v1, rows_v0, rows_v1,
              out_v, pos_v, segtab_v, gam_v, bet_v, gsem0, gsem1,
              csem0, csem1, ssem):
    wid = lax.axis_index("s") * NC + lax.axis_index("c")
    comb_v = [comb_v0, comb_v1]
    pidx_v = [pidx_v0, pidx_v1]
    rows_v = [rows_v0, rows_v1]
    gsem = [gsem0, gsem1]
    csem = [csem0, csem1]

    pltpu.sync_copy(pos_hbm, pos_v.at[pl.ds(0, SEQ)])
    pltpu.sync_copy(pos_hbm, pos_v.at[pl.ds(SEQ, SEQ)])
    pltpu.sync_copy(segtab_hbm, segtab_v)
    pltpu.sync_copy(gam_hbm, gam_v)
    pltpu.sync_copy(bet_hbm, bet_v)

    def fold(t, carry):
        for j in range(NJ):
            sl = pl.ds(j * L, L)
            pos_v[t, sl] = pos_v[t, sl] + segtab_v[0, sl]
            pos_v[SEQ + t, sl] = pos_v[SEQ + t, sl] + segtab_v[1, sl]
        return carry

    lax.fori_loop(0, SEQ, fold, 0)

    gam = [gam_v[pl.ds(j * L, L)] for j in range(NJ)]
    bet = [bet_v[pl.ds(j * L, L)] for j in range(NJ)]

    inv_dim = jnp.float32(1.0 / DIM)
    zeros16 = jnp.zeros((L,), jnp.int32)
    iota16 = lax.iota(jnp.int32, L)

    NG = 2
    GR = CHUNK // NG

    def stage_comb(k, b):
        return pltpu.async_copy(comb_hbm.at[wid * CPW + k], comb_v[b],
                                csem[b])

    def calc_pidx(k, b):
        base = (wid * CPW + k) * CHUNK
        for g in range(CHUNK // L):
            sl = pl.ds(g * L, L)
            sg = comb_v[b][1, sl]
            pidx_v[b][sl] = lax.rem(base + g * L + iota16, SEQ) + sg * SEQ

    def fire_gather(k, b):
        return [
            pltpu.async_copy(
                tok_hbm.at[comb_v[b].at[0, pl.ds(h * GR, GR)]],
                rows_v[b].at[pl.ds(h * GR, GR)], gsem[b])
            for h in range(NG)
        ]

    def compute_chunk(k, b):
        c = wid * CPW + k
        base = c * CHUNK

        NU = 4
        NQ = 4
        QTOK = CHUNK // NQ
        ones16 = zeros16 + 1
        stores = []
        for q in range(NQ):
            qbase = q * QTOK

            def tok_body(i, tc, qbase=qbase):
                ts = [qbase + NU * i + u for u in range(NU)]
                xs = [[] for _ in range(NU)]
                stats = []
                for u in range(NU):
                    t = ts[u]
                    p = plsc.load_gather(pidx_v[b], [zeros16 + t])[0]
                    acc = None
                    acc2 = None
                    for j in range(NJ):
                        sl = pl.ds(j * L, L)
                        x = rows_v[b][t, sl] + pos_v[p, sl]
                        xs[u].append(x)
                        acc = x if acc is None else acc + x
                        acc2 = x * x if acc2 is None else acc2 + x * x
                    stats.append((acc, acc2))
                rm = []
                for u in range(NU):
                    acc, acc2 = stats[u]
                    mean = jnp.sum(acc) * inv_dim
                    var = jnp.maximum(
                        jnp.sum(acc2) * inv_dim - mean * mean, 0.0)
                    rvec = _rsqrt(
                        jnp.full((L,), var, jnp.float32) + jnp.float32(EPS))
                    mvec = jnp.full((L,), mean, jnp.float32)
                    rm.append((rvec, mvec))
                for u in range(NU):
                    rvec, mvec = rm[u]
                    for j in range(NJ):
                        sl = pl.ds(j * L, L)
                        out_v[ts[u], sl] = (
                            (xs[u][j] - mvec) * rvec * gam[j] + bet[j])
                return tc

            lax.fori_loop(0, QTOK // NU, tok_body, 0)
            stores.append(pltpu.async_copy(
                out_v.at[pl.ds(qbase, QTOK)],
                out_hbm.at[c, pl.ds(qbase, QTOK)], ssem))
        for d in stores:
            d.wait()

    stage_comb(0, 0).wait()
    calc_pidx(0, 0)
    dc1 = stage_comb(1, 1)
    g0 = fire_gather(0, 0)
    dc1.wait()
    calc_pidx(1, 1)
    for d in g0:
        d.wait()

    def pair_body(i, carry):
        k = 2 * i
        dc = stage_comb(k + 2, 0)
        g1 = fire_gather(k + 1, 1)
        compute_chunk(k, 0)
        dc.wait()
        calc_pidx(k + 2, 0)
        for d in g1:
            d.wait()
        dc = stage_comb(k + 3, 1)
        g0b = fire_gather(k + 2, 0)
        compute_chunk(k + 1, 1)
        dc.wait()
        calc_pidx(k + 3, 1)
        for d in g0b:
            d.wait()
        return carry

    lax.fori_loop(0, CPW // 2 - 1, pair_body, 0)
    g1 = fire_gather(CPW - 1, 1)
    compute_chunk(CPW - 2, 0)
    for d in g1:
        d.wait()
    compute_chunk(CPW - 1, 1)


@jax.jit
def _emb_call(comb, token_table, pos_table, seg_table, gamma, beta):
    mesh = plsc.VectorSubcoreMesh(core_axis_name="c", subcore_axis_name="s",
                                  num_cores=NC, num_subcores=NS)
    fn = pl.kernel(
        _emb_body,
        out_type=jax.ShapeDtypeStruct((NCHUNKS, CHUNK, DIM), jnp.float32),
        mesh=mesh,
        compiler_params=pltpu.CompilerParams(needs_layout_passes=False),
        scratch_types=[
            pltpu.VMEM((2, CHUNK), jnp.int32),
            pltpu.VMEM((2, CHUNK), jnp.int32),
            pltpu.VMEM((CHUNK,), jnp.int32),
            pltpu.VMEM((CHUNK,), jnp.int32),
            pltpu.VMEM((CHUNK, DIM), jnp.float32),
            pltpu.VMEM((CHUNK, DIM), jnp.float32),
            pltpu.VMEM((CHUNK, DIM), jnp.float32),
            pltpu.VMEM((2 * SEQ, DIM), jnp.float32),
            pltpu.VMEM((2, DIM), jnp.float32),
            pltpu.VMEM((DIM,), jnp.float32),
            pltpu.VMEM((DIM,), jnp.float32),
            pltpu.SemaphoreType.DMA,
            pltpu.SemaphoreType.DMA,
            pltpu.SemaphoreType.DMA,
            pltpu.SemaphoreType.DMA,
            pltpu.SemaphoreType.DMA,
        ],
    )
    return fn(comb, token_table, pos_table, seg_table, gamma, beta)


def kernel(input_ids, segment_ids, token_table, pos_table, seg_table, gamma, beta):
    ids2 = input_ids.reshape(NCHUNKS, CHUNK).astype(jnp.int32)
    seg2 = segment_ids.reshape(NCHUNKS, CHUNK).astype(jnp.int32)
    comb = jnp.stack([ids2, seg2], axis=1)
    out = _emb_call(comb, token_table, pos_table, seg_table, gamma, beta)
    return out.reshape(BATCH, SEQ, DIM)


# speedup vs baseline: 1.1648x; 1.0028x over previous
import functools

import jax
import jax.numpy as jnp
from jax import lax
from jax.experimental import pallas as pl
from jax.experimental.pallas import tpu as pltpu
from jax.experimental.pallas import tpu_sc as plsc

VOCAB = 100000
DIM = 128
SEQ = 200
BATCH = 1024

NC = 2
NS = 16
NW = NC * NS
L = 16
NJ = DIM // L

TOTAL = BATCH * SEQ
CHUNK = 128
NCHUNKS = TOTAL // CHUNK
CPW = NCHUNKS // NW
EPS = 1e-12


def _rsqrt(x):
    i = plsc.bitcast(x, jnp.int32)
    i = jnp.int32(0x5F3759DF) - (i >> 1)
    y = plsc.bitcast(i, jnp.float32)
    half = x * jnp.float32(0.5)
    for _ in range(2):
        y = y * (jnp.float32(1.5) - half * y * y)
    return y


def _emb_body(comb_hbm, tok_hbm, pos_hbm, segtab_hbm, gam_hbm, bet_hbm,
              out_hbm, comb_v0, comb_v1, pidx_v0, pidx_v1, rows_v0, rows_v1,
              out_v, pos_v, segtab_v, gam_v, bet_v, gsem0, gsem1,
              csem0, csem1, ssem):
    wid = lax.axis_index("s") * NC + lax.axis_index("c")
    comb_v = [comb_v0, comb_v1]
    pidx_v = [pidx_v0, pidx_v1]
    rows_v = [rows_v0, rows_v1]
    gsem = [gsem0, gsem1]
    csem = [csem0, csem1]

    pltpu.sync_copy(pos_hbm, pos_v.at[pl.ds(0, SEQ)])
    pltpu.sync_copy(pos_hbm, pos_v.at[pl.ds(SEQ, SEQ)])
    pltpu.sync_copy(segtab_hbm, segtab_v)
    pltpu.sync_copy(gam_hbm, gam_v)
    pltpu.sync_copy(bet_hbm, bet_v)

    def fold(t, carry):
        for j in range(NJ):
            sl = pl.ds(j * L, L)
            pos_v[t, sl] = pos_v[t, sl] + segtab_v[0, sl]
            pos_v[SEQ + t, sl] = pos_v[SEQ + t, sl] + segtab_v[1, sl]
        return carry

    lax.fori_loop(0, SEQ, fold, 0)

    gam = [gam_v[pl.ds(j * L, L)] for j in range(NJ)]
    bet = [bet_v[pl.ds(j * L, L)] for j in range(NJ)]

    inv_dim = jnp.float32(1.0 / DIM)
    zeros16 = jnp.zeros((L,), jnp.int32)
    iota16 = lax.iota(jnp.int32, L)

    NG = 2
    GR = CHUNK // NG

    def stage_comb(k, b):
        return pltpu.async_copy(comb_hbm.at[wid * CPW + k], comb_v[b],
                                csem[b])

    def calc_pidx(k, b):
        base = (wid * CPW + k) * CHUNK
        for g in range(CHUNK // L):
            sl = pl.ds(g * L, L)
            sg = comb_v[b][1, sl]
            pidx_v[b][sl] = lax.rem(base + g * L + iota16, SEQ) + sg * SEQ

    def fire_gather(k, b):
        return [
            pltpu.async_copy(
                tok_hbm.at[comb_v[b].at[0, pl.ds(h * GR, GR)]],
                rows_v[b].at[pl.ds(h * GR, GR)], gsem[b])
            for h in range(NG)
        ]

    def compute_chunk(k, b):
        c = wid * CPW + k
        base = c * CHUNK

        NU = 4
        NQ = 2
        QTOK = CHUNK // NQ
        ones16 = zeros16 + 1
        stores = []
        for q in range(NQ):
            qbase = q * QTOK

            def tok_body(i, tc, qbase=qbase):
                ts = [qbase + NU * i + u for u in range(NU)]
                xs = [[] for _ in range(NU)]
                stats = []
                for u in range(NU):
                    t = ts[u]
                    p = plsc.load_gather(pidx_v[b], [zeros16 + t])[0]
                    acc = None
                    acc2 = None
                    for j in range(NJ):
                        sl = pl.ds(j * L, L)
                        x = rows_v[b][t, sl] + pos_v[p, sl]
                        xs[u].append(x)
                        acc = x if acc is None else acc + x
                        acc2 = x * x if acc2 is None else acc2 + x * x
                    stats.append((acc, acc2))
                rm = []
                for u in range(NU):
                    acc, acc2 = stats[u]
                    mean = jnp.sum(acc) * inv_dim
                    var = jnp.maximum(
                        jnp.sum(acc2) * inv_dim - mean * mean, 0.0)
                    rvec = _rsqrt(
                        jnp.full((L,), var, jnp.float32) + jnp.float32(EPS))
                    mvec = jnp.full((L,), mean, jnp.float32)
                    rm.append((rvec, mvec))
                for u in range(NU):
                    rvec, mvec = rm[u]
                    for j in range(NJ):
                        sl = pl.ds(j * L, L)
                        out_v[ts[u], sl] = (
                            (xs[u][j] - mvec) * rvec * gam[j] + bet[j])
                return tc

            lax.fori_loop(0, QTOK // NU, tok_body, 0)
            stores.append(pltpu.async_copy(
                out_v.at[pl.ds(qbase, QTOK)],
                out_hbm.at[c, pl.ds(qbase, QTOK)], ssem))
        for d in stores:
            d.wait()

    stage_comb(0, 0).wait()
    calc_pidx(0, 0)
    dc1 = stage_comb(1, 1)
    g0 = fire_gather(0, 0)
    dc1.wait()
    calc_pidx(1, 1)
    for d in g0:
        d.wait()

    def pair_body(i, carry):
        k = 2 * i
        dc = stage_comb(k + 2, 0)
        g1 = fire_gather(k + 1, 1)
        compute_chunk(k, 0)
        dc.wait()
        calc_pidx(k + 2, 0)
        for d in g1:
            d.wait()
        dc = stage_comb(k + 3, 1)
        g0b = fire_gather(k + 2, 0)
        compute_chunk(k + 1, 1)
        dc.wait()
        calc_pidx(k + 3, 1)
        for d in g0b:
            d.wait()
        return carry

    lax.fori_loop(0, CPW // 2 - 1, pair_body, 0)
    g1 = fire_gather(CPW - 1, 1)
    compute_chunk(CPW - 2, 0)
    for d in g1:
        d.wait()
    compute_chunk(CPW - 1, 1)


@jax.jit
def _emb_call(comb, token_table, pos_table, seg_table, gamma, beta):
    mesh = plsc.VectorSubcoreMesh(core_axis_name="c", subcore_axis_name="s",
                                  num_cores=NC, num_subcores=NS)
    fn = pl.kernel(
        _emb_body,
        out_type=jax.ShapeDtypeStruct((NCHUNKS, CHUNK, DIM), jnp.float32),
        mesh=mesh,
        compiler_params=pltpu.CompilerParams(needs_layout_passes=False),
        scratch_types=[
            pltpu.VMEM((2, CHUNK), jnp.int32),
            pltpu.VMEM((2, CHUNK), jnp.int32),
            pltpu.VMEM((CHUNK,), jnp.int32),
            pltpu.VMEM((CHUNK,), jnp.int32),
            pltpu.VMEM((CHUNK, DIM), jnp.float32),
            pltpu.VMEM((CHUNK, DIM), jnp.float32),
            pltpu.VMEM((CHUNK, DIM), jnp.float32),
            pltpu.VMEM((2 * SEQ, DIM), jnp.float32),
            pltpu.VMEM((2, DIM), jnp.float32),
            pltpu.VMEM((DIM,), jnp.float32),
            pltpu.VMEM((DIM,), jnp.float32),
            pltpu.SemaphoreType.DMA,
            pltpu.SemaphoreType.DMA,
            pltpu.SemaphoreType.DMA,
            pltpu.SemaphoreType.DMA,
            pltpu.SemaphoreType.DMA,
        ],
    )
    return fn(comb, token_table, pos_table, seg_table, gamma, beta)


def kernel(input_ids, segment_ids, token_table, pos_table, seg_table, gamma, beta):
    ids2 = input_ids.reshape(NCHUNKS, CHUNK).astype(jnp.int32)
    seg2 = segment_ids.reshape(NCHUNKS, CHUNK).astype(jnp.int32)
    comb = jnp.stack([ids2, seg2], axis=1)
    out = _emb_call(comb, token_table, pos_table, seg_table, gamma, beta)
    return out.reshape(BATCH, SEQ, DIM)
